# baseline (speedup 1.0000x reference)
import functools

import jax
import jax.numpy as jnp
from jax import lax
from jax.experimental import pallas as pl
from jax.experimental.pallas import tpu as pltpu
from jax.experimental.pallas import tpu_sc as plsc

U = 10000
I = 10000
D = 128
NNZ = 320000

NC = 2
NS = 16
NW = NC * NS

EB = 128
NBT = 80
NBLKP = NBT * NW
PAD = NBLKP * EB - NNZ
SB = 40
ROWS_PER_TILE = 624


def _scale_block(gbuf, valsv, b):
    def group(g, carry):
        vv = valsv[b, pl.ds(g * 16, 16)]
        for l in range(16):
            v = vv[l]
            e = g * 16 + l
            for j in range(D // 16):
                gbuf[e, pl.ds(j * 16, 16)] = gbuf[e, pl.ds(j * 16, 16)] * v
        return carry

    lax.fori_loop(0, EB // 16, group, 0)


def _zero_rows(buf):
    def zrow(r, carry):
        for j in range(D // 16):
            buf[r, pl.ds(j * 16, 16)] = jnp.zeros((16,), jnp.float32)
        return carry

    lax.fori_loop(0, 128, zrow, 0)


def _spmm_accumulate(rows2, cols2, vals2, x_hbm, out_hbm,
                     acc, colsv, ridxv, valsv, gb, gs, ss, c, s):
    wid = s * NC + c
    startblk = wid * NBT
    rbase = s * ROWS_PER_TILE
    last_off = jnp.where(s == NS - 1, 512, 496)

    _zero_rows(gb[0])
    for off in (0, 128, 256, 384):
        pltpu.sync_copy(gb[0], acc.at[pl.ds(rbase + off, 128)])
    pltpu.sync_copy(gb[0], acc.at[pl.ds(rbase + last_off, 128)])
    plsc.subcore_barrier()

    for h in range(2):
        sb0 = startblk + h * SB
        pltpu.sync_copy(cols2.at[pl.ds(sb0, SB)], colsv)
        pltpu.sync_copy(rows2.at[pl.ds(sb0, SB)], ridxv)
        pltpu.sync_copy(vals2.at[pl.ds(sb0, SB)], valsv)

        pltpu.async_copy(x_hbm.at[colsv.at[0]], gb[0], gs[0])

        def duo(k, carry):
            for j in range(2):
                b = 2 * k + j
                nj = 1 - j

                @pl.when((b >= 1) & (b + 1 < SB))
                def _refill():
                    pltpu.make_async_copy(
                        gb[nj], acc.at[ridxv.at[b - 1]], ss[nj]).wait()
                    pltpu.async_copy(x_hbm.at[colsv.at[b + 1]], gb[nj], gs[nj])

                if j == 0:
                    @pl.when(b < 1)
                    def _prime():
                        pltpu.async_copy(x_hbm.at[colsv.at[1]], gb[1], gs[1])

                pltpu.make_async_copy(
                    x_hbm.at[colsv.at[b]], gb[j], gs[j]).wait()
                _scale_block(gb[j], valsv, b)
                pltpu.async_copy(gb[j], acc.at[ridxv.at[b]], ss[j], add=True)
            return carry

        lax.fori_loop(0, SB // 2, duo, 0)
        for i in range(2):
            bb = SB - 2 + i
            pltpu.make_async_copy(gb[bb % 2], acc.at[ridxv.at[bb]],
                                  ss[bb % 2]).wait()

    plsc.subcore_barrier()

    for off in (0, 128, 256, 384):
        pltpu.sync_copy(acc.at[pl.ds(rbase + off, 128)],
                        out_hbm.at[c, pl.ds(rbase + off, 128)])
    pltpu.sync_copy(acc.at[pl.ds(rbase + last_off, 128)],
                    out_hbm.at[c, pl.ds(rbase + last_off, 128)])
    plsc.subcore_barrier()


_SC_SCRATCH = [
    pltpu.VMEM_SHARED((U, D), jnp.float32),
    pltpu.VMEM((SB, EB), jnp.int32),
    pltpu.VMEM((SB, EB), jnp.int32),
    pltpu.VMEM((SB, EB), jnp.float32),
] + [pltpu.VMEM((EB, D), jnp.float32)] * 2 \
  + [pltpu.SemaphoreType.DMA] * 4

_MESH = plsc.VectorSubcoreMesh(core_axis_name="c", subcore_axis_name="s")
_SC_PARAMS = pltpu.CompilerParams()


@functools.partial(
    pl.kernel,
    out_type=[jax.ShapeDtypeStruct((NC, U, D), jnp.float32)] * 4,
    mesh=_MESH,
    scratch_types=_SC_SCRATCH,
    compiler_params=_SC_PARAMS,
)
def _phase_a(eu, ei, ui_r, ui_c, ui_v, iu_r, iu_c, iu_v, u3_r, u3_c, u3_v,
             p_g1u, p_g1i, p_g3u, p_g3i,
             acc, colsv, ridxv, valsv, g0, g1, gsem0, gsem1, ssem0, ssem1):
    gb, gs, ss = (g0, g1), (gsem0, gsem1), (ssem0, ssem1)
    c = lax.axis_index("c")
    s = lax.axis_index("s")
    args = (acc, colsv, ridxv, valsv, gb, gs, ss, c, s)
    _spmm_accumulate(ui_r, ui_c, ui_v, ei, p_g1u, *args)
    _spmm_accumulate(iu_r, iu_c, iu_v, eu, p_g1i, *args)
    _spmm_accumulate(u3_r, u3_c, u3_v, ei, p_g3u, *args)
    _spmm_accumulate(u3_c, u3_r, u3_v, eu, p_g3i, *args)


@functools.partial(
    pl.kernel,
    out_type=[jax.ShapeDtypeStruct((NC, U, D), jnp.float32)] * 2,
    mesh=_MESH,
    scratch_types=_SC_SCRATCH,
    compiler_params=_SC_PARAMS,
)
def _phase_b(g1u, g1i, ui_r, ui_c, ui_v, iu_r, iu_c, iu_v,
             p_g2u, p_g2i,
             acc, colsv, ridxv, valsv, g0, g1, gsem0, gsem1, ssem0, ssem1):
    gb, gs, ss = (g0, g1), (gsem0, gsem1), (ssem0, ssem1)
    c = lax.axis_index("c")
    s = lax.axis_index("s")
    args = (acc, colsv, ridxv, valsv, gb, gs, ss, c, s)
    _spmm_accumulate(ui_r, ui_c, ui_v, g1i, p_g2u, *args)
    _spmm_accumulate(iu_r, iu_c, iu_v, g1u, p_g2i, *args)


_RB = 1000
_GRID = U // _RB


def _combine1_body(p1u, p1i, g1u, g1i):
    g1u[...] = p1u[0] + p1u[1]
    g1i[...] = p1i[0] + p1i[1]


def _combine1(p_g1u, p_g1i):
    return pl.pallas_call(
        _combine1_body,
        grid=(_GRID,),
        in_specs=[pl.BlockSpec((NC, _RB, D), lambda i: (0, i, 0))] * 2,
        out_specs=[pl.BlockSpec((_RB, D), lambda i: (i, 0))] * 2,
        out_shape=[jax.ShapeDtypeStruct((U, D), jnp.float32)] * 2,
    )(p_g1u, p_g1i)


def _combine2_body(eu, ei, g1u, g1i, p2u, p2i, p3u, p3i, ujs, ou, oi):
    g3u = p3u[0] + p3u[1]
    ou[...] = 0.25 * (eu[...] + g1u[...] + (p2u[0] + p2u[1])) + g3u * ujs[...]
    oi[...] = 0.25 * (ei[...] + g1i[...] + (p2i[0] + p2i[1])
                      + (p3i[0] + p3i[1]))


def _combine2(eu, ei, g1u, g1i, p_g2u, p_g2i, p_g3u, p_g3i, user_js):
    dense = pl.BlockSpec((_RB, D), lambda i: (i, 0))
    part = pl.BlockSpec((NC, _RB, D), lambda i: (0, i, 0))
    return pl.pallas_call(
        _combine2_body,
        grid=(_GRID,),
        in_specs=[dense, dense, dense, dense, part, part, part, part,
                  pl.BlockSpec((_RB, 1), lambda i: (i, 0))],
        out_specs=[dense, dense],
        out_shape=[jax.ShapeDtypeStruct((U, D), jnp.float32)] * 2,
    )(eu, ei, g1u, g1i, p_g2u, p_g2i, p_g3u, p_g3i, user_js)


def kernel(embed_user, embed_item, ui_vals, iu_vals, ui3_vals, user_js,
           ui_rows, ui_cols, iu_rows, iu_cols, ui3_rows, ui3_cols):
    zi = jnp.zeros((PAD,), jnp.int32)
    zf = jnp.zeros((PAD,), jnp.float32)

    def blki(a):
        return jnp.concatenate([a.astype(jnp.int32), zi]).reshape(NBLKP, EB)

    def blkf(a):
        return jnp.concatenate([a, zf]).reshape(NBLKP, EB)

    ui_r, ui_c, ui_v = blki(ui_rows), blki(ui_cols), blkf(ui_vals)
    iu_r, iu_c, iu_v = blki(iu_rows), blki(iu_cols), blkf(iu_vals)
    u3_r, u3_c, u3_v = blki(ui3_rows), blki(ui3_cols), blkf(ui3_vals)

    p_g1u, p_g1i, p_g3u, p_g3i = _phase_a(
        embed_user, embed_item,
        ui_r, ui_c, ui_v, iu_r, iu_c, iu_v, u3_r, u3_c, u3_v)
    g1u, g1i = _combine1(p_g1u, p_g1i)
    p_g2u, p_g2i = _phase_b(g1u, g1i, ui_r, ui_c, ui_v, iu_r, iu_c, iu_v)
    return _combine2(embed_user, embed_item, g1u, g1i,
                     p_g2u, p_g2i, p_g3u, p_g3i, user_js)


# speedup vs baseline: 1.0006x; 1.0006x over previous
import functools

import jax
import jax.numpy as jnp
from jax import lax
from jax.experimental import pallas as pl
from jax.experimental.pallas import tpu as pltpu
from jax.experimental.pallas import tpu_sc as plsc

U = 10000
I = 10000
D = 128
NNZ = 320000

NC = 2
NS = 16
NW = NC * NS

EB = 128
NBT = 80
NBLKP = NBT * NW
PAD = NBLKP * EB - NNZ
SB = 40
ROWS_PER_TILE = 624


def _scale_block(gbuf, valsv, b):
    def group(g, carry):
        vv = valsv[b, pl.ds(g * 16, 16)]
        for l in range(16):
            v = vv[l]
            e = g * 16 + l
            for j in range(D // 16):
                gbuf[e, pl.ds(j * 16, 16)] = gbuf[e, pl.ds(j * 16, 16)] * v
        return carry

    lax.fori_loop(0, EB // 16, group, 0)


def _zero_rows(buf):
    def zrow(r, carry):
        for j in range(D // 16):
            buf[r, pl.ds(j * 16, 16)] = jnp.zeros((16,), jnp.float32)
        return carry

    lax.fori_loop(0, 128, zrow, 0)


def _spmm_accumulate(rows2, cols2, vals2, x_hbm, out_hbm,
                     acc, colsv, ridxv, valsv, gb, gs, ss, c, s):
    wid = s * NC + c
    startblk = wid * NBT
    rbase = s * ROWS_PER_TILE
    last_off = jnp.where(s == NS - 1, 512, 496)

    _zero_rows(gb[0])
    for off in (0, 128, 256, 384):
        pltpu.sync_copy(gb[0], acc.at[pl.ds(rbase + off, 128)])
    pltpu.sync_copy(gb[0], acc.at[pl.ds(rbase + last_off, 128)])
    plsc.subcore_barrier()

    for h in range(2):
        sb0 = startblk + h * SB
        pltpu.sync_copy(cols2.at[pl.ds(sb0, SB)], colsv)
        pltpu.sync_copy(rows2.at[pl.ds(sb0, SB)], ridxv)
        pltpu.sync_copy(vals2.at[pl.ds(sb0, SB)], valsv)

        pltpu.async_copy(x_hbm.at[colsv.at[0]], gb[0], gs[0])

        def duo(k, carry):
            for j in range(2):
                b = 2 * k + j
                nj = 1 - j
                pltpu.make_async_copy(
                    x_hbm.at[colsv.at[b]], gb[j], gs[j]).wait()

                @pl.when(b + 1 < SB)
                def _prefetch():
                    pltpu.async_copy(x_hbm.at[colsv.at[b + 1]], gb[nj], gs[nj])

                _scale_block(gb[j], valsv, b)
                pltpu.sync_copy(gb[j], acc.at[ridxv.at[b]], add=True)
            return carry

        lax.fori_loop(0, SB // 2, duo, 0)

    plsc.subcore_barrier()

    for off in (0, 128, 256, 384):
        pltpu.sync_copy(acc.at[pl.ds(rbase + off, 128)],
                        out_hbm.at[c, pl.ds(rbase + off, 128)])
    pltpu.sync_copy(acc.at[pl.ds(rbase + last_off, 128)],
                    out_hbm.at[c, pl.ds(rbase + last_off, 128)])
    plsc.subcore_barrier()


_SC_SCRATCH = [
    pltpu.VMEM_SHARED((U, D), jnp.float32),
    pltpu.VMEM((SB, EB), jnp.int32),
    pltpu.VMEM((SB, EB), jnp.int32),
    pltpu.VMEM((SB, EB), jnp.float32),
] + [pltpu.VMEM((EB, D), jnp.float32)] * 2 \
  + [pltpu.SemaphoreType.DMA] * 4

_MESH = plsc.VectorSubcoreMesh(core_axis_name="c", subcore_axis_name="s")
_SC_PARAMS = pltpu.CompilerParams()


@functools.partial(
    pl.kernel,
    out_type=[jax.ShapeDtypeStruct((NC, U, D), jnp.float32)] * 4,
    mesh=_MESH,
    scratch_types=_SC_SCRATCH,
    compiler_params=_SC_PARAMS,
)
def _phase_a(eu, ei, ui_r, ui_c, ui_v, iu_r, iu_c, iu_v, u3_r, u3_c, u3_v,
             p_g1u, p_g1i, p_g3u, p_g3i,
             acc, colsv, ridxv, valsv, g0, g1, gsem0, gsem1, ssem0, ssem1):
    gb, gs, ss = (g0, g1), (gsem0, gsem1), (ssem0, ssem1)
    c = lax.axis_index("c")
    s = lax.axis_index("s")
    args = (acc, colsv, ridxv, valsv, gb, gs, ss, c, s)
    _spmm_accumulate(ui_r, ui_c, ui_v, ei, p_g1u, *args)
    _spmm_accumulate(iu_r, iu_c, iu_v, eu, p_g1i, *args)
    _spmm_accumulate(u3_r, u3_c, u3_v, ei, p_g3u, *args)
    _spmm_accumulate(u3_c, u3_r, u3_v, eu, p_g3i, *args)


@functools.partial(
    pl.kernel,
    out_type=[jax.ShapeDtypeStruct((NC, U, D), jnp.float32)] * 2,
    mesh=_MESH,
    scratch_types=_SC_SCRATCH,
    compiler_params=_SC_PARAMS,
)
def _phase_b(g1u, g1i, ui_r, ui_c, ui_v, iu_r, iu_c, iu_v,
             p_g2u, p_g2i,
             acc, colsv, ridxv, valsv, g0, g1, gsem0, gsem1, ssem0, ssem1):
    gb, gs, ss = (g0, g1), (gsem0, gsem1), (ssem0, ssem1)
    c = lax.axis_index("c")
    s = lax.axis_index("s")
    args = (acc, colsv, ridxv, valsv, gb, gs, ss, c, s)
    _spmm_accumulate(ui_r, ui_c, ui_v, g1i, p_g2u, *args)
    _spmm_accumulate(iu_r, iu_c, iu_v, g1u, p_g2i, *args)


_RB = 1000
_GRID = U // _RB


def _combine1_body(p1u, p1i, g1u, g1i):
    g1u[...] = p1u[0] + p1u[1]
    g1i[...] = p1i[0] + p1i[1]


def _combine1(p_g1u, p_g1i):
    return pl.pallas_call(
        _combine1_body,
        grid=(_GRID,),
        in_specs=[pl.BlockSpec((NC, _RB, D), lambda i: (0, i, 0))] * 2,
        out_specs=[pl.BlockSpec((_RB, D), lambda i: (i, 0))] * 2,
        out_shape=[jax.ShapeDtypeStruct((U, D), jnp.float32)] * 2,
    )(p_g1u, p_g1i)


def _combine2_body(eu, ei, g1u, g1i, p2u, p2i, p3u, p3i, ujs, ou, oi):
    g3u = p3u[0] + p3u[1]
    ou[...] = 0.25 * (eu[...] + g1u[...] + (p2u[0] + p2u[1])) + g3u * ujs[...]
    oi[...] = 0.25 * (ei[...] + g1i[...] + (p2i[0] + p2i[1])
                      + (p3i[0] + p3i[1]))


def _combine2(eu, ei, g1u, g1i, p_g2u, p_g2i, p_g3u, p_g3i, user_js):
    dense = pl.BlockSpec((_RB, D), lambda i: (i, 0))
    part = pl.BlockSpec((NC, _RB, D), lambda i: (0, i, 0))
    return pl.pallas_call(
        _combine2_body,
        grid=(_GRID,),
        in_specs=[dense, dense, dense, dense, part, part, part, part,
                  pl.BlockSpec((_RB, 1), lambda i: (i, 0))],
        out_specs=[dense, dense],
        out_shape=[jax.ShapeDtypeStruct((U, D), jnp.float32)] * 2,
    )(eu, ei, g1u, g1i, p_g2u, p_g2i, p_g3u, p_g3i, user_js)


def kernel(embed_user, embed_item, ui_vals, iu_vals, ui3_vals, user_js,
           ui_rows, ui_cols, iu_rows, iu_cols, ui3_rows, ui3_cols):
    zi = jnp.zeros((PAD,), jnp.int32)
    zf = jnp.zeros((PAD,), jnp.float32)

    def blki(a):
        return jnp.concatenate([a.astype(jnp.int32), zi]).reshape(NBLKP, EB)

    def blkf(a):
        return jnp.concatenate([a, zf]).reshape(NBLKP, EB)

    ui_r, ui_c, ui_v = blki(ui_rows), blki(ui_cols), blkf(ui_vals)
    iu_r, iu_c, iu_v = blki(iu_rows), blki(iu_cols), blkf(iu_vals)
    u3_r, u3_c, u3_v = blki(ui3_rows), blki(ui3_cols), blkf(ui3_vals)

    p_g1u, p_g1i, p_g3u, p_g3i = _phase_a(
        embed_user, embed_item,
        ui_r, ui_c, ui_v, iu_r, iu_c, iu_v, u3_r, u3_c, u3_v)
    g1u, g1i = _combine1(p_g1u, p_g1i)
    p_g2u, p_g2i = _phase_b(g1u, g1i, ui_r, ui_c, ui_v, iu_r, iu_c, iu_v)
    return _combine2(embed_user, embed_item, g1u, g1i,
                     p_g2u, p_g2i, p_g3u, p_g3i, user_js)


# speedup vs baseline: 3.2344x; 3.2325x over previous
import functools

import jax
import jax.numpy as jnp
from jax import lax
from jax.experimental import pallas as pl
from jax.experimental.pallas import tpu as pltpu
from jax.experimental.pallas import tpu_sc as plsc

U = 10000
I = 10000
D = 128
NNZ = 320000

NC = 2
NS = 16
NW = NC * NS

EB = 128
NBT = 80
NBLKP = NBT * NW
PAD = NBLKP * EB - NNZ
SB = 40
ROWS_PER_TILE = 624


def _scale_block(gbuf, valsv, b):
    def group(g, carry):
        vv = valsv[b, pl.ds(g * 16, 16)]
        for l in range(16):
            v = vv[l]
            e = g * 16 + l
            for j in range(D // 16):
                gbuf[e, pl.ds(j * 16, 16)] = gbuf[e, pl.ds(j * 16, 16)] * v
        return carry

    lax.fori_loop(0, EB // 16, group, 0)


def _zero_rows(buf):
    def zrow(r, carry):
        for j in range(D // 16):
            buf[r, pl.ds(j * 16, 16)] = jnp.zeros((16,), jnp.float32)
        return carry

    lax.fori_loop(0, 128, zrow, 0)


def _spmm_accumulate(rows2, cols2, vals2, x_hbm, out_hbm,
                     acc, colsv, ridxv, valsv, gb, gs, ss, c, s):
    wid = s * NC + c
    startblk = wid * NBT
    rbase = s * ROWS_PER_TILE
    last_off = jnp.where(s == NS - 1, 512, 496)

    _zero_rows(gb[0])
    for off in (0, 128, 256, 384):
        pltpu.sync_copy(gb[0], acc.at[pl.ds(rbase + off, 128)])
    pltpu.sync_copy(gb[0], acc.at[pl.ds(rbase + last_off, 128)])
    plsc.subcore_barrier()

    for h in range(2):
        sb0 = startblk + h * SB
        pltpu.sync_copy(cols2.at[pl.ds(sb0, SB)], colsv)
        pltpu.sync_copy(rows2.at[pl.ds(sb0, SB)], ridxv)
        pltpu.sync_copy(vals2.at[pl.ds(sb0, SB)], valsv)

        pltpu.async_copy(x_hbm.at[colsv.at[0]], gb[0], gs[0])

        def duo(k, carry):
            for j in range(2):
                b = 2 * k + j
                nj = 1 - j
                pltpu.make_async_copy(
                    x_hbm.at[colsv.at[b]], gb[j], gs[j]).wait()

                @pl.when(b + 1 < SB)
                def _prefetch():
                    pltpu.async_copy(x_hbm.at[colsv.at[b + 1]], gb[nj], gs[nj])

                _scale_block(gb[j], valsv, b)
                pltpu.sync_copy(gb[j], acc.at[ridxv.at[b]], add=True)
            return carry

        lax.fori_loop(0, SB // 2, duo, 0)

    plsc.subcore_barrier()

    for off in (0, 128, 256, 384):
        pltpu.sync_copy(acc.at[pl.ds(rbase + off, 128)],
                        out_hbm.at[c, pl.ds(rbase + off, 128)])
    pltpu.sync_copy(acc.at[pl.ds(rbase + last_off, 128)],
                    out_hbm.at[c, pl.ds(rbase + last_off, 128)])
    plsc.subcore_barrier()


_SC_SCRATCH = [
    pltpu.VMEM_SHARED((U, D), jnp.float32),
    pltpu.VMEM((SB, EB), jnp.int32),
    pltpu.VMEM((SB, EB), jnp.int32),
    pltpu.VMEM((SB, EB), jnp.float32),
] + [pltpu.VMEM((EB, D), jnp.float32)] * 2 \
  + [pltpu.SemaphoreType.DMA] * 4

_MESH = plsc.VectorSubcoreMesh(core_axis_name="c", subcore_axis_name="s")
_SC_PARAMS = pltpu.CompilerParams()


@functools.partial(
    pl.kernel,
    out_type=[jax.ShapeDtypeStruct((NC, U, D), jnp.float32)] * 4,
    mesh=_MESH,
    scratch_types=_SC_SCRATCH,
    compiler_params=_SC_PARAMS,
)
def _phase_a(eu, ei, ui_r, ui_c, ui_v, iu_r, iu_c, iu_v, u3_r, u3_c, u3_v,
             p_g1u, p_g1i, p_g3u, p_g3i,
             acc, colsv, ridxv, valsv, g0, g1, gsem0, gsem1, ssem0, ssem1):
    gb, gs, ss = (g0, g1), (gsem0, gsem1), (ssem0, ssem1)
    c = lax.axis_index("c")
    s = lax.axis_index("s")
    args = (acc, colsv, ridxv, valsv, gb, gs, ss, c, s)
    _spmm_accumulate(ui_r, ui_c, ui_v, ei, p_g1u, *args)
    _spmm_accumulate(iu_r, iu_c, iu_v, eu, p_g1i, *args)
    _spmm_accumulate(u3_r, u3_c, u3_v, ei, p_g3u, *args)
    _spmm_accumulate(u3_c, u3_r, u3_v, eu, p_g3i, *args)


@functools.partial(
    pl.kernel,
    out_type=[jax.ShapeDtypeStruct((NC, U, D), jnp.float32)] * 2,
    mesh=_MESH,
    scratch_types=_SC_SCRATCH,
    compiler_params=_SC_PARAMS,
)
def _phase_b(g1u, g1i, ui_r, ui_c, ui_v, iu_r, iu_c, iu_v,
             p_g2u, p_g2i,
             acc, colsv, ridxv, valsv, g0, g1, gsem0, gsem1, ssem0, ssem1):
    gb, gs, ss = (g0, g1), (gsem0, gsem1), (ssem0, ssem1)
    c = lax.axis_index("c")
    s = lax.axis_index("s")
    args = (acc, colsv, ridxv, valsv, gb, gs, ss, c, s)
    _spmm_accumulate(ui_r, ui_c, ui_v, g1i, p_g2u, *args)
    _spmm_accumulate(iu_r, iu_c, iu_v, g1u, p_g2i, *args)


_RB = 1000
_GRID = U // _RB


def _combine1_body(p1u, p1i, g1u, g1i):
    g1u[...] = p1u[0] + p1u[1]
    g1i[...] = p1i[0] + p1i[1]


def _combine1(p_g1u, p_g1i):
    return pl.pallas_call(
        _combine1_body,
        grid=(_GRID,),
        in_specs=[pl.BlockSpec((NC, _RB, D), lambda i: (0, i, 0))] * 2,
        out_specs=[pl.BlockSpec((_RB, D), lambda i: (i, 0))] * 2,
        out_shape=[jax.ShapeDtypeStruct((U, D), jnp.float32)] * 2,
    )(p_g1u, p_g1i)


def _combine2_body(eu, ei, g1u, g1i, p2u, p2i, p3u, p3i, ujs, ou, oi):
    g3u = p3u[0] + p3u[1]
    ou[...] = 0.25 * (eu[...] + g1u[...] + (p2u[0] + p2u[1])) + g3u * ujs[...]
    oi[...] = 0.25 * (ei[...] + g1i[...] + (p2i[0] + p2i[1])
                      + (p3i[0] + p3i[1]))


def _combine2(eu, ei, g1u, g1i, p_g2u, p_g2i, p_g3u, p_g3i, user_js):
    dense = pl.BlockSpec((_RB, D), lambda i: (i, 0))
    part = pl.BlockSpec((NC, _RB, D), lambda i: (0, i, 0))
    return pl.pallas_call(
        _combine2_body,
        grid=(_GRID,),
        in_specs=[dense, dense, dense, dense, part, part, part, part,
                  pl.BlockSpec((_RB, 1), lambda i: (i, 0))],
        out_specs=[dense, dense],
        out_shape=[jax.ShapeDtypeStruct((U, D), jnp.float32)] * 2,
    )(eu, ei, g1u, g1i, p_g2u, p_g2i, p_g3u, p_g3i, user_js)


def kernel(embed_user, embed_item, ui_vals, iu_vals, ui3_vals, user_js,
           ui_rows, ui_cols, iu_rows, iu_cols, ui3_rows, ui3_cols):
    zi = jnp.arange(PAD, dtype=jnp.int32) % U
    zf = jnp.zeros((PAD,), jnp.float32)

    def blki(a):
        return jnp.concatenate([a.astype(jnp.int32), zi]).reshape(NBLKP, EB)

    def blkf(a):
        return jnp.concatenate([a, zf]).reshape(NBLKP, EB)

    ui_r, ui_c, ui_v = blki(ui_rows), blki(ui_cols), blkf(ui_vals)
    iu_r, iu_c, iu_v = blki(iu_rows), blki(iu_cols), blkf(iu_vals)
    u3_r, u3_c, u3_v = blki(ui3_rows), blki(ui3_cols), blkf(ui3_vals)

    p_g1u, p_g1i, p_g3u, p_g3i = _phase_a(
        embed_user, embed_item,
        ui_r, ui_c, ui_v, iu_r, iu_c, iu_v, u3_r, u3_c, u3_v)
    g1u, g1i = _combine1(p_g1u, p_g1i)
    p_g2u, p_g2i = _phase_b(g1u, g1i, ui_r, ui_c, ui_v, iu_r, iu_c, iu_v)
    return _combine2(embed_user, embed_item, g1u, g1i,
                     p_g2u, p_g2i, p_g3u, p_g3i, user_js)


# speedup vs baseline: 3.2476x; 1.0041x over previous
import functools

import jax
import jax.numpy as jnp
from jax import lax
from jax.experimental import pallas as pl
from jax.experimental.pallas import tpu as pltpu
from jax.experimental.pallas import tpu_sc as plsc

U = 10000
I = 10000
D = 128
NNZ = 320000

NC = 2
NS = 16
NW = NC * NS

EB = 128
NBT = 80
NBLKP = NBT * NW
PAD = NBLKP * EB - NNZ
SB = 40
ROWS_PER_TILE = 624


def _scale_block(gbuf, valsv, b):
    def group(g, carry):
        vv = valsv[b, pl.ds(g * 16, 16)]
        for l in range(16):
            v = vv[l]
            e = g * 16 + l
            for j in range(D // 16):
                gbuf[e, pl.ds(j * 16, 16)] = gbuf[e, pl.ds(j * 16, 16)] * v
        return carry

    lax.fori_loop(0, EB // 16, group, 0)


def _zero_rows(buf):
    def zrow(r, carry):
        for j in range(D // 16):
            buf[r, pl.ds(j * 16, 16)] = jnp.zeros((16,), jnp.float32)
        return carry

    lax.fori_loop(0, 128, zrow, 0)


def _spmm_accumulate(rows2, cols2, vals2, x_hbm, out_hbm,
                     acc, colsv, ridxv, valsv, gb, gs, ss, c, s):
    wid = s * NC + c
    startblk = wid * NBT
    rbase = s * ROWS_PER_TILE
    last_off = jnp.where(s == NS - 1, 512, 496)

    _zero_rows(gb[0])
    for off in (0, 128, 256, 384):
        pltpu.sync_copy(gb[0], acc.at[pl.ds(rbase + off, 128)])
    pltpu.sync_copy(gb[0], acc.at[pl.ds(rbase + last_off, 128)])
    plsc.subcore_barrier()

    for h in range(2):
        sb0 = startblk + h * SB
        pltpu.sync_copy(cols2.at[pl.ds(sb0, SB)], colsv)
        pltpu.sync_copy(rows2.at[pl.ds(sb0, SB)], ridxv)
        pltpu.sync_copy(vals2.at[pl.ds(sb0, SB)], valsv)

        pltpu.async_copy(x_hbm.at[colsv.at[0]], gb[0], gs[0])

        def duo(k, carry):
            for j in range(2):
                b = 2 * k + j
                nj = 1 - j

                @pl.when((b >= 1) & (b + 1 < SB))
                def _refill():
                    pltpu.make_async_copy(
                        gb[nj], acc.at[ridxv.at[b - 1]], ss[nj]).wait()
                    pltpu.async_copy(x_hbm.at[colsv.at[b + 1]], gb[nj], gs[nj])

                if j == 0:
                    @pl.when(b < 1)
                    def _prime():
                        pltpu.async_copy(x_hbm.at[colsv.at[1]], gb[1], gs[1])

                pltpu.make_async_copy(
                    x_hbm.at[colsv.at[b]], gb[j], gs[j]).wait()
                _scale_block(gb[j], valsv, b)
                pltpu.async_copy(gb[j], acc.at[ridxv.at[b]], ss[j], add=True)
            return carry

        lax.fori_loop(0, SB // 2, duo, 0)
        for i in range(2):
            bb = SB - 2 + i
            pltpu.make_async_copy(gb[bb % 2], acc.at[ridxv.at[bb]],
                                  ss[bb % 2]).wait()

    plsc.subcore_barrier()

    for off in (0, 128, 256, 384):
        pltpu.sync_copy(acc.at[pl.ds(rbase + off, 128)],
                        out_hbm.at[c, pl.ds(rbase + off, 128)])
    pltpu.sync_copy(acc.at[pl.ds(rbase + last_off, 128)],
                    out_hbm.at[c, pl.ds(rbase + last_off, 128)])
    plsc.subcore_barrier()


_SC_SCRATCH = [
    pltpu.VMEM_SHARED((U, D), jnp.float32),
    pltpu.VMEM((SB, EB), jnp.int32),
    pltpu.VMEM((SB, EB), jnp.int32),
    pltpu.VMEM((SB, EB), jnp.float32),
] + [pltpu.VMEM((EB, D), jnp.float32)] * 2 \
  + [pltpu.SemaphoreType.DMA] * 4

_MESH = plsc.VectorSubcoreMesh(core_axis_name="c", subcore_axis_name="s")
_SC_PARAMS = pltpu.CompilerParams()


@functools.partial(
    pl.kernel,
    out_type=[jax.ShapeDtypeStruct((NC, U, D), jnp.float32)] * 4,
    mesh=_MESH,
    scratch_types=_SC_SCRATCH,
    compiler_params=_SC_PARAMS,
)
def _phase_a(eu, ei, ui_r, ui_c, ui_v, iu_r, iu_c, iu_v, u3_r, u3_c, u3_v,
             p_g1u, p_g1i, p_g3u, p_g3i,
             acc, colsv, ridxv, valsv, g0, g1, gsem0, gsem1, ssem0, ssem1):
    gb, gs, ss = (g0, g1), (gsem0, gsem1), (ssem0, ssem1)
    c = lax.axis_index("c")
    s = lax.axis_index("s")
    args = (acc, colsv, ridxv, valsv, gb, gs, ss, c, s)
    _spmm_accumulate(ui_r, ui_c, ui_v, ei, p_g1u, *args)
    _spmm_accumulate(iu_r, iu_c, iu_v, eu, p_g1i, *args)
    _spmm_accumulate(u3_r, u3_c, u3_v, ei, p_g3u, *args)
    _spmm_accumulate(u3_c, u3_r, u3_v, eu, p_g3i, *args)


@functools.partial(
    pl.kernel,
    out_type=[jax.ShapeDtypeStruct((NC, U, D), jnp.float32)] * 2,
    mesh=_MESH,
    scratch_types=_SC_SCRATCH,
    compiler_params=_SC_PARAMS,
)
def _phase_b(g1u, g1i, ui_r, ui_c, ui_v, iu_r, iu_c, iu_v,
             p_g2u, p_g2i,
             acc, colsv, ridxv, valsv, g0, g1, gsem0, gsem1, ssem0, ssem1):
    gb, gs, ss = (g0, g1), (gsem0, gsem1), (ssem0, ssem1)
    c = lax.axis_index("c")
    s = lax.axis_index("s")
    args = (acc, colsv, ridxv, valsv, gb, gs, ss, c, s)
    _spmm_accumulate(ui_r, ui_c, ui_v, g1i, p_g2u, *args)
    _spmm_accumulate(iu_r, iu_c, iu_v, g1u, p_g2i, *args)


_RB = 1000
_GRID = U // _RB


def _combine1_body(p1u, p1i, g1u, g1i):
    g1u[...] = p1u[0] + p1u[1]
    g1i[...] = p1i[0] + p1i[1]


def _combine1(p_g1u, p_g1i):
    return pl.pallas_call(
        _combine1_body,
        grid=(_GRID,),
        in_specs=[pl.BlockSpec((NC, _RB, D), lambda i: (0, i, 0))] * 2,
        out_specs=[pl.BlockSpec((_RB, D), lambda i: (i, 0))] * 2,
        out_shape=[jax.ShapeDtypeStruct((U, D), jnp.float32)] * 2,
    )(p_g1u, p_g1i)


def _combine2_body(eu, ei, g1u, g1i, p2u, p2i, p3u, p3i, ujs, ou, oi):
    g3u = p3u[0] + p3u[1]
    ou[...] = 0.25 * (eu[...] + g1u[...] + (p2u[0] + p2u[1])) + g3u * ujs[...]
    oi[...] = 0.25 * (ei[...] + g1i[...] + (p2i[0] + p2i[1])
                      + (p3i[0] + p3i[1]))


def _combine2(eu, ei, g1u, g1i, p_g2u, p_g2i, p_g3u, p_g3i, user_js):
    dense = pl.BlockSpec((_RB, D), lambda i: (i, 0))
    part = pl.BlockSpec((NC, _RB, D), lambda i: (0, i, 0))
    return pl.pallas_call(
        _combine2_body,
        grid=(_GRID,),
        in_specs=[dense, dense, dense, dense, part, part, part, part,
                  pl.BlockSpec((_RB, 1), lambda i: (i, 0))],
        out_specs=[dense, dense],
        out_shape=[jax.ShapeDtypeStruct((U, D), jnp.float32)] * 2,
    )(eu, ei, g1u, g1i, p_g2u, p_g2i, p_g3u, p_g3i, user_js)


def kernel(embed_user, embed_item, ui_vals, iu_vals, ui3_vals, user_js,
           ui_rows, ui_cols, iu_rows, iu_cols, ui3_rows, ui3_cols):
    zi = jnp.arange(PAD, dtype=jnp.int32) % U
    zf = jnp.zeros((PAD,), jnp.float32)

    def blki(a):
        return jnp.concatenate([a.astype(jnp.int32), zi]).reshape(NBLKP, EB)

    def blkf(a):
        return jnp.concatenate([a, zf]).reshape(NBLKP, EB)

    ui_r, ui_c, ui_v = blki(ui_rows), blki(ui_cols), blkf(ui_vals)
    iu_r, iu_c, iu_v = blki(iu_rows), blki(iu_cols), blkf(iu_vals)
    u3_r, u3_c, u3_v = blki(ui3_rows), blki(ui3_cols), blkf(ui3_vals)

    p_g1u, p_g1i, p_g3u, p_g3i = _phase_a(
        embed_user, embed_item,
        ui_r, ui_c, ui_v, iu_r, iu_c, iu_v, u3_r, u3_c, u3_v)
    g1u, g1i = _combine1(p_g1u, p_g1i)
    p_g2u, p_g2i = _phase_b(g1u, g1i, ui_r, ui_c, ui_v, iu_r, iu_c, iu_v)
    return _combine2(embed_user, embed_item, g1u, g1i,
                     p_g2u, p_g2i, p_g3u, p_g3i, user_js)
